# Initial kernel scaffold; baseline (speedup 1.0000x reference)
import jax
import jax.numpy as jnp
from jax.experimental import pallas as pl


def kernel(loc_data, conf_data, priors, targets):
    raise NotImplementedError("write your pallas kernel here")


# speedup vs baseline: 37.1382x; 37.1382x over previous
import jax
import jax.numpy as jnp
from jax.experimental import pallas as pl

_THRESHOLD = 0.35
_VAR0, _VAR1 = 0.1, 0.2
_NEG_RATIO = 3
_B, _P, _NO = 16, 25600, 32


def _match_body(pt_ref, tgt_ref, loc_ref, conf_ref, ce_ref, part_ref):
    pt = pt_ref[...]
    cx, cy, w, h = pt[0:1, :], pt[1:2, :], pt[2:3, :], pt[3:4, :]
    x0, y0 = cx - w * 0.5, cy - h * 0.5
    x1, y1 = cx + w * 0.5, cy + h * 0.5

    tg = tgt_ref[0]
    tx0, ty0 = tg[:, 0:1], tg[:, 1:2]
    tx1, ty1 = tg[:, 2:3], tg[:, 3:4]

    iw = jnp.maximum(jnp.minimum(tx1, x1) - jnp.maximum(tx0, x0), 0.0)
    ih = jnp.maximum(jnp.minimum(ty1, y1) - jnp.maximum(ty0, y0), 0.0)
    inter = iw * ih
    area_a = (tx1 - tx0) * (ty1 - ty0)
    area_b = w * h
    ov = inter / (area_a + area_b - inter)

    sub_iota = jax.lax.broadcasted_iota(jnp.int32, (_NO, _P), 0)
    lane_iota = jax.lax.broadcasted_iota(jnp.int32, (_NO, _P), 1)

    bto = jnp.max(ov, axis=0, keepdims=True)
    bti = jnp.min(jnp.where(ov == bto, sub_iota, _NO), axis=0, keepdims=True)
    bpo = jnp.max(ov, axis=1, keepdims=True)
    bpi = jnp.min(jnp.where(ov == bpo, lane_iota, _P), axis=1, keepdims=True)

    eqf = lane_iota == bpi
    f_idx = jnp.max(jnp.where(eqf, sub_iota, -1), axis=0, keepdims=True)
    bti = jnp.where(f_idx >= 0, f_idx, bti)
    bto = jnp.where(f_idx >= 0, 2.0, bto)
    pos = bto >= _THRESHOLD
    posf = pos.astype(jnp.float32)

    eq2 = (sub_iota == bti).astype(jnp.float32)
    mx0 = jnp.sum(eq2 * tx0, axis=0, keepdims=True)
    my0 = jnp.sum(eq2 * ty0, axis=0, keepdims=True)
    mx1 = jnp.sum(eq2 * tx1, axis=0, keepdims=True)
    my1 = jnp.sum(eq2 * ty1, axis=0, keepdims=True)

    g_cx = ((mx0 + mx1) * 0.5 - cx) / (_VAR0 * w)
    g_cy = ((my0 + my1) * 0.5 - cy) / (_VAR0 * h)
    g_w = jnp.log((mx1 - mx0) / w) / _VAR1
    g_h = jnp.log((my1 - my0) / h) / _VAR1

    ld = loc_ref[0]

    def _sl1(d):
        ad = jnp.abs(d)
        return jnp.where(ad < 1.0, 0.5 * d * d, ad - 0.5)

    sl1 = (_sl1(ld[0:1, :] - g_cx) + _sl1(ld[1:2, :] - g_cy)
           + _sl1(ld[2:3, :] - g_w) + _sl1(ld[3:4, :] - g_h))
    sl1_sum = jnp.sum(sl1 * posf)
    npos = jnp.sum(posf)

    cd = conf_ref[0]
    c0, c1 = cd[0:1, :], cd[1:2, :]
    dng = jnp.where(pos, c0 - c1, c1 - c0)
    ce = jnp.maximum(dng, 0.0) + jnp.log1p(jnp.exp(-jnp.abs(dng)))
    pce = jnp.sum(ce * posf)

    ce_ref[0] = jnp.where(pos, -1.0, ce)

    li = jax.lax.broadcasted_iota(jnp.int32, (1, 128), 1)
    part_ref[0] = (jnp.where(li == 0, sl1_sum, 0.0)
                   + jnp.where(li == 1, npos, 0.0)
                   + jnp.where(li == 2, pce, 0.0))


def _select_body(ce_ref, part_ref, out_loc_ref, out_conf_ref):
    ce = ce_ref[:, 0, :]
    part = part_ref[:, 0, :]
    sl1 = part[:, 0:1]
    nposf = part[:, 1:2]
    pce = part[:, 2:3]

    s_total = jnp.sum(nposf)
    k = jnp.minimum(jnp.minimum(_NEG_RATIO * nposf, float(_P - 1)),
                    float(_P) - nposf)

    ci = jax.lax.bitcast_convert_type(ce, jnp.int32)
    lo = jnp.zeros((_B, 1), jnp.int32)
    hi = jnp.max(ci, axis=1, keepdims=True) + 1

    def body(_, carry):
        lo, hi = carry
        mid = lo + jax.lax.div(hi - lo, 2)
        cnt = jnp.sum(jnp.where(ci >= mid, 1.0, 0.0), axis=1, keepdims=True)
        ok = cnt >= k
        return jnp.where(ok, mid, lo), jnp.where(ok, hi, mid)

    lo, hi = jax.lax.fori_loop(0, 31, body, (lo, hi))
    t = lo
    tf = jax.lax.bitcast_convert_type(t, jnp.float32)
    gt = ci > t
    cnt_gt = jnp.sum(jnp.where(gt, 1.0, 0.0), axis=1, keepdims=True)
    sum_gt = jnp.sum(jnp.where(gt, ce, 0.0), axis=1, keepdims=True)
    neg_sum = sum_gt + (k - cnt_gt) * tf

    total_ce = jnp.sum(pce) + jnp.sum(neg_sum)
    total_sel = s_total + jnp.sum(k)
    out_loc_ref[...] = jnp.reshape(jnp.sum(sl1) / (4.0 * s_total) / s_total,
                                   (1, 1))
    out_conf_ref[...] = jnp.reshape(total_ce / total_sel / s_total, (1, 1))


def kernel(loc_data, conf_data, priors, targets):
    pt = priors.T
    loc_t = jnp.transpose(loc_data, (0, 2, 1))
    conf_t = jnp.transpose(conf_data, (0, 2, 1))

    ce, part = pl.pallas_call(
        _match_body,
        grid=(_B,),
        in_specs=[
            pl.BlockSpec((4, _P), lambda i: (0, 0)),
            pl.BlockSpec((1, _NO, 5), lambda i: (i, 0, 0)),
            pl.BlockSpec((1, 4, _P), lambda i: (i, 0, 0)),
            pl.BlockSpec((1, 2, _P), lambda i: (i, 0, 0)),
        ],
        out_specs=[
            pl.BlockSpec((1, 1, _P), lambda i: (i, 0, 0)),
            pl.BlockSpec((1, 1, 128), lambda i: (i, 0, 0)),
        ],
        out_shape=[
            jax.ShapeDtypeStruct((_B, 1, _P), jnp.float32),
            jax.ShapeDtypeStruct((_B, 1, 128), jnp.float32),
        ],
    )(pt, targets, loc_t, conf_t)

    out_loc, out_conf = pl.pallas_call(
        _select_body,
        out_shape=[
            jax.ShapeDtypeStruct((1, 1), jnp.float32),
            jax.ShapeDtypeStruct((1, 1), jnp.float32),
        ],
    )(ce, part)

    return out_loc[0, 0], out_conf[0, 0]


# speedup vs baseline: 51.4237x; 1.3847x over previous
import jax
import jax.numpy as jnp
from jax.experimental import pallas as pl
from jax.experimental.pallas import tpu as pltpu

_THRESHOLD = 0.35
_VAR0, _VAR1 = 0.1, 0.2
_NEG_RATIO = 3
_B, _P, _NO = 16, 25600, 32
_PS, _PL = 200, 128


def _match_body(pt_ref, tgt_ref, loc_ref, conf_ref, ce_ref, part_ref):
    cx, cy, w, h = pt_ref[0], pt_ref[1], pt_ref[2], pt_ref[3]
    x0, y0 = cx - 0.5 * w, cy - 0.5 * h
    x1, y1 = cx + 0.5 * w, cy + 0.5 * h
    area_b = w * h
    pid = (jax.lax.broadcasted_iota(jnp.int32, (_PS, _PL), 0) * _PL
           + jax.lax.broadcasted_iota(jnp.int32, (_PS, _PL), 1))

    bto = jnp.full((_PS, _PL), -1.0, jnp.float32)
    bti = jnp.zeros((_PS, _PL), jnp.int32)
    f_idx = jnp.full((_PS, _PL), -1, jnp.int32)

    ovs = []
    for j in range(_NO):
        tx0 = tgt_ref[0, j, 0]
        ty0 = tgt_ref[0, j, 1]
        tx1 = tgt_ref[0, j, 2]
        ty1 = tgt_ref[0, j, 3]
        iw = jnp.maximum(jnp.minimum(tx1, x1) - jnp.maximum(tx0, x0), 0.0)
        ih = jnp.maximum(jnp.minimum(ty1, y1) - jnp.maximum(ty0, y0), 0.0)
        inter = iw * ih
        aa = (tx1 - tx0) * (ty1 - ty0)
        ov = inter / (aa + area_b - inter)
        ovs.append(ov)
        better = ov > bto
        bti = jnp.where(better, j, bti)
        bto = jnp.where(better, ov, bto)

    for j in range(_NO):
        ov = ovs[j]
        bpo = jnp.max(ov)
        bpi = jnp.min(jnp.where(ov == bpo, pid, _P))
        f_idx = jnp.where(pid == bpi, j, f_idx)

    forced = f_idx >= 0
    bti = jnp.where(forced, f_idx, bti)
    bto = jnp.where(forced, 2.0, bto)
    pos = bto >= _THRESHOLD
    posf = pos.astype(jnp.float32)

    msx = jnp.zeros((_PS, _PL), jnp.float32)
    mdx = jnp.zeros((_PS, _PL), jnp.float32)
    msy = jnp.zeros((_PS, _PL), jnp.float32)
    mdy = jnp.zeros((_PS, _PL), jnp.float32)
    for j in range(_NO):
        eq = bti == j
        msx = jnp.where(eq, tgt_ref[0, j, 0] + tgt_ref[0, j, 2], msx)
        mdx = jnp.where(eq, tgt_ref[0, j, 2] - tgt_ref[0, j, 0], mdx)
        msy = jnp.where(eq, tgt_ref[0, j, 1] + tgt_ref[0, j, 3], msy)
        mdy = jnp.where(eq, tgt_ref[0, j, 3] - tgt_ref[0, j, 1], mdy)

    inv_w, inv_h = 1.0 / w, 1.0 / h
    g_cx = (0.5 * msx - cx) * ((1.0 / _VAR0) * inv_w)
    g_cy = (0.5 * msy - cy) * ((1.0 / _VAR0) * inv_h)
    g_w = jnp.log(mdx * inv_w) * (1.0 / _VAR1)
    g_h = jnp.log(mdy * inv_h) * (1.0 / _VAR1)

    def _sl1(d):
        ad = jnp.abs(d)
        return jnp.where(ad < 1.0, 0.5 * d * d, ad - 0.5)

    sl1 = (_sl1(loc_ref[0, 0] - g_cx) + _sl1(loc_ref[0, 1] - g_cy)
           + _sl1(loc_ref[0, 2] - g_w) + _sl1(loc_ref[0, 3] - g_h))
    sl1_sum = jnp.sum(sl1 * posf)
    npos = jnp.sum(posf)

    c0, c1 = conf_ref[0, 0], conf_ref[0, 1]
    dng = jnp.where(pos, c0 - c1, c1 - c0)
    ce = jnp.maximum(dng, 0.0) + jnp.log1p(jnp.exp(-jnp.abs(dng)))
    pce = jnp.sum(ce * posf)

    ce_ref[...] = jnp.where(pos, -1.0, ce)[None]

    li = jax.lax.broadcasted_iota(jnp.int32, (1, 1, 128), 2)
    part_ref[...] = (jnp.where(li == 0, sl1_sum, 0.0)
                     + jnp.where(li == 1, npos, 0.0)
                     + jnp.where(li == 2, pce, 0.0))


def _select_body(ce_ref, part_ref, out_loc_ref, out_conf_ref):
    ce = ce_ref[...]
    part = part_ref[...]
    sl1 = part[:, :, 0:1]
    nposf = part[:, :, 1:2]
    pce = part[:, :, 2:3]

    s_total = jnp.sum(nposf)
    k = jnp.minimum(jnp.minimum(_NEG_RATIO * nposf, float(_P - 1)),
                    float(_P) - nposf)

    def _rowsum(x):
        return jnp.sum(jnp.sum(x, axis=1, keepdims=True), axis=2,
                       keepdims=True)

    ci = jax.lax.bitcast_convert_type(ce, jnp.int32)
    lo = jnp.zeros((_B, 1, 1), jnp.int32)
    hi = jnp.max(jnp.max(ci, axis=1, keepdims=True), axis=2,
                 keepdims=True) + 1

    def body(_, carry):
        lo, hi = carry
        mid = lo + jax.lax.div(hi - lo, 2)
        cnt = _rowsum(jnp.where(ci >= mid, 1.0, 0.0))
        ok = cnt >= k
        return jnp.where(ok, mid, lo), jnp.where(ok, hi, mid)

    lo, hi = jax.lax.fori_loop(0, 31, body, (lo, hi))
    t = lo
    tf = jax.lax.bitcast_convert_type(t, jnp.float32)
    gt = ci > t
    cnt_gt = _rowsum(jnp.where(gt, 1.0, 0.0))
    sum_gt = _rowsum(jnp.where(gt, ce, 0.0))
    neg_sum = sum_gt + (k - cnt_gt) * tf

    total_ce = jnp.sum(pce) + jnp.sum(neg_sum)
    total_sel = s_total + jnp.sum(k)
    out_loc_ref[...] = jnp.reshape(jnp.sum(sl1) / (4.0 * s_total) / s_total,
                                   (1, 1))
    out_conf_ref[...] = jnp.reshape(total_ce / total_sel / s_total, (1, 1))


def kernel(loc_data, conf_data, priors, targets):
    pt = priors.T.reshape(4, _PS, _PL)
    loc_t = jnp.transpose(loc_data, (0, 2, 1)).reshape(_B, 4, _PS, _PL)
    conf_t = jnp.transpose(conf_data, (0, 2, 1)).reshape(_B, 2, _PS, _PL)

    ce, part = pl.pallas_call(
        _match_body,
        grid=(_B,),
        in_specs=[
            pl.BlockSpec((4, _PS, _PL), lambda i: (0, 0, 0)),
            pl.BlockSpec((1, _NO, 5), lambda i: (i, 0, 0),
                         memory_space=pltpu.SMEM),
            pl.BlockSpec((1, 4, _PS, _PL), lambda i: (i, 0, 0, 0)),
            pl.BlockSpec((1, 2, _PS, _PL), lambda i: (i, 0, 0, 0)),
        ],
        out_specs=[
            pl.BlockSpec((1, _PS, _PL), lambda i: (i, 0, 0)),
            pl.BlockSpec((1, 1, 128), lambda i: (i, 0, 0)),
        ],
        out_shape=[
            jax.ShapeDtypeStruct((_B, _PS, _PL), jnp.float32),
            jax.ShapeDtypeStruct((_B, 1, 128), jnp.float32),
        ],
    )(pt, targets, loc_t, conf_t)

    out_loc, out_conf = pl.pallas_call(
        _select_body,
        out_shape=[
            jax.ShapeDtypeStruct((1, 1), jnp.float32),
            jax.ShapeDtypeStruct((1, 1), jnp.float32),
        ],
    )(ce, part)

    return out_loc[0, 0], out_conf[0, 0]


# speedup vs baseline: 59.7498x; 1.1619x over previous
import jax
import jax.numpy as jnp
from jax.experimental import pallas as pl
from jax.experimental.pallas import tpu as pltpu

_THRESHOLD = 0.35
_VAR0, _VAR1 = 0.1, 0.2
_NEG_RATIO = 3
_B, _P, _NO = 16, 25600, 32
_PS, _PL = 200, 128


def _match_body(pt_ref, tgt_ref, loc_ref, conf_ref, ce_ref, part_ref):
    cx, cy, w, h = pt_ref[0], pt_ref[1], pt_ref[2], pt_ref[3]
    x0, y0 = cx - 0.5 * w, cy - 0.5 * h
    x1, y1 = cx + 0.5 * w, cy + 0.5 * h
    area_b = w * h
    pid = (jax.lax.broadcasted_iota(jnp.int32, (_PS, _PL), 0) * _PL
           + jax.lax.broadcasted_iota(jnp.int32, (_PS, _PL), 1))

    bto = jnp.full((_PS, _PL), -1.0, jnp.float32)
    bti = jnp.zeros((_PS, _PL), jnp.int32)
    f_idx = jnp.full((_PS, _PL), -1, jnp.int32)

    for j in range(_NO):
        tx0 = tgt_ref[0, j, 0]
        ty0 = tgt_ref[0, j, 1]
        tx1 = tgt_ref[0, j, 2]
        ty1 = tgt_ref[0, j, 3]
        iw = jnp.maximum(jnp.minimum(tx1, x1) - jnp.maximum(tx0, x0), 0.0)
        ih = jnp.maximum(jnp.minimum(ty1, y1) - jnp.maximum(ty0, y0), 0.0)
        inter = iw * ih
        aa = (tx1 - tx0) * (ty1 - ty0)
        ov = inter / (aa + area_b - inter)
        better = ov > bto
        bti = jnp.where(better, j, bti)
        bto = jnp.where(better, ov, bto)
        bpo = jnp.max(ov)
        bpi = jnp.min(jnp.where(ov == bpo, pid, _P))
        f_idx = jnp.where(pid == bpi, j, f_idx)

    forced = f_idx >= 0
    bti = jnp.where(forced, f_idx, bti)
    bto = jnp.where(forced, 2.0, bto)
    pos = bto >= _THRESHOLD
    posf = pos.astype(jnp.float32)

    msx = jnp.zeros((_PS, _PL), jnp.float32)
    mdx = jnp.zeros((_PS, _PL), jnp.float32)
    msy = jnp.zeros((_PS, _PL), jnp.float32)
    mdy = jnp.zeros((_PS, _PL), jnp.float32)
    for j in range(_NO):
        eq = bti == j
        msx = jnp.where(eq, tgt_ref[0, j, 0] + tgt_ref[0, j, 2], msx)
        mdx = jnp.where(eq, tgt_ref[0, j, 2] - tgt_ref[0, j, 0], mdx)
        msy = jnp.where(eq, tgt_ref[0, j, 1] + tgt_ref[0, j, 3], msy)
        mdy = jnp.where(eq, tgt_ref[0, j, 3] - tgt_ref[0, j, 1], mdy)

    inv_w, inv_h = 1.0 / w, 1.0 / h
    g_cx = (0.5 * msx - cx) * ((1.0 / _VAR0) * inv_w)
    g_cy = (0.5 * msy - cy) * ((1.0 / _VAR0) * inv_h)
    g_w = jnp.log(mdx * inv_w) * (1.0 / _VAR1)
    g_h = jnp.log(mdy * inv_h) * (1.0 / _VAR1)

    def _sl1(d):
        ad = jnp.abs(d)
        return jnp.where(ad < 1.0, 0.5 * d * d, ad - 0.5)

    sl1 = (_sl1(loc_ref[0, 0] - g_cx) + _sl1(loc_ref[0, 1] - g_cy)
           + _sl1(loc_ref[0, 2] - g_w) + _sl1(loc_ref[0, 3] - g_h))
    sl1_sum = jnp.sum(sl1 * posf)
    npos = jnp.sum(posf)

    c0, c1 = conf_ref[0, 0], conf_ref[0, 1]
    dng = jnp.where(pos, c0 - c1, c1 - c0)
    ce = jnp.maximum(dng, 0.0) + jnp.log1p(jnp.exp(-jnp.abs(dng)))
    pce = jnp.sum(ce * posf)

    ce_ref[...] = jnp.where(pos, -1.0, ce)[None]

    li = jax.lax.broadcasted_iota(jnp.int32, (1, 1, 128), 2)
    part_ref[...] = (jnp.where(li == 0, sl1_sum, 0.0)
                     + jnp.where(li == 1, npos, 0.0)
                     + jnp.where(li == 2, pce, 0.0))


def _select_body(ce_ref, part_ref, out_loc_ref, out_conf_ref):
    ce = ce_ref[...]
    part = part_ref[...]
    sl1 = part[:, :, 0:1]
    nposf = part[:, :, 1:2]
    pce = part[:, :, 2:3]

    s_total = jnp.sum(nposf)
    k = jnp.minimum(jnp.minimum(_NEG_RATIO * nposf, float(_P - 1)),
                    float(_P) - nposf)

    def _rowsum(x):
        return jnp.sum(jnp.sum(x, axis=1, keepdims=True), axis=2,
                       keepdims=True)

    ci = jax.lax.bitcast_convert_type(ce, jnp.int32)
    lo = jnp.zeros((_B, 1, 1), jnp.int32)
    hi = jnp.max(jnp.max(ci, axis=1, keepdims=True), axis=2,
                 keepdims=True) + 1

    def body(_, carry):
        lo, hi = carry
        mid = lo + jax.lax.div(hi - lo, 2)
        cnt = _rowsum(jnp.where(ci >= mid, 1.0, 0.0))
        ok = cnt >= k
        return jnp.where(ok, mid, lo), jnp.where(ok, hi, mid)

    lo, hi = jax.lax.fori_loop(0, 31, body, (lo, hi))
    t = lo
    tf = jax.lax.bitcast_convert_type(t, jnp.float32)
    gt = ci > t
    cnt_gt = _rowsum(jnp.where(gt, 1.0, 0.0))
    sum_gt = _rowsum(jnp.where(gt, ce, 0.0))
    neg_sum = sum_gt + (k - cnt_gt) * tf

    total_ce = jnp.sum(pce) + jnp.sum(neg_sum)
    total_sel = s_total + jnp.sum(k)
    out_loc_ref[...] = jnp.reshape(jnp.sum(sl1) / (4.0 * s_total) / s_total,
                                   (1, 1))
    out_conf_ref[...] = jnp.reshape(total_ce / total_sel / s_total, (1, 1))


def kernel(loc_data, conf_data, priors, targets):
    pt = priors.T.reshape(4, _PS, _PL)
    loc_t = jnp.transpose(loc_data, (0, 2, 1)).reshape(_B, 4, _PS, _PL)
    conf_t = jnp.transpose(conf_data, (0, 2, 1)).reshape(_B, 2, _PS, _PL)

    ce, part = pl.pallas_call(
        _match_body,
        grid=(_B,),
        in_specs=[
            pl.BlockSpec((4, _PS, _PL), lambda i: (0, 0, 0)),
            pl.BlockSpec((1, _NO, 5), lambda i: (i, 0, 0),
                         memory_space=pltpu.SMEM),
            pl.BlockSpec((1, 4, _PS, _PL), lambda i: (i, 0, 0, 0)),
            pl.BlockSpec((1, 2, _PS, _PL), lambda i: (i, 0, 0, 0)),
        ],
        out_specs=[
            pl.BlockSpec((1, _PS, _PL), lambda i: (i, 0, 0)),
            pl.BlockSpec((1, 1, 128), lambda i: (i, 0, 0)),
        ],
        out_shape=[
            jax.ShapeDtypeStruct((_B, _PS, _PL), jnp.float32),
            jax.ShapeDtypeStruct((_B, 1, 128), jnp.float32),
        ],
    )(pt, targets, loc_t, conf_t)

    out_loc, out_conf = pl.pallas_call(
        _select_body,
        out_shape=[
            jax.ShapeDtypeStruct((1, 1), jnp.float32),
            jax.ShapeDtypeStruct((1, 1), jnp.float32),
        ],
    )(ce, part)

    return out_loc[0, 0], out_conf[0, 0]
